# baseline (speedup 1.0000x reference)
import dataclasses
import functools
import jax
import jax.numpy as jnp
from jax import lax
from jax.experimental import pallas as pl
from jax.experimental.pallas import tpu as pltpu
from jax.experimental.pallas import tpu_sc as plsc

_BATCH = 16
_N = 32
_S = 2048
_L = 2048
_D = 768
_C = 256
_NCH = _L // _C
_LANES = 16


def _sc_body(att_hbm, st_hbm, en_hbm, out_hbm, att_v, st_v, en_v, res_v,
             sem):
    cid = lax.axis_index("c")
    b = lax.axis_index("s")

    @pl.when(cid == 0)
    def _():
        pltpu.async_copy(att_hbm.at[pl.ds(b * _S, _S)], att_v, sem).wait()
        pltpu.async_copy(st_hbm.at[pl.ds(b * _N, _N)],
                         st_v.at[pl.ds(0, _N)], sem).wait()
        pltpu.async_copy(en_hbm.at[pl.ds(b * _N, _N)],
                         en_v.at[pl.ds(0, _N)], sem).wait()
        lane = lax.broadcasted_iota(jnp.int32, (_LANES,), 0)

        def sent_body(n, carry):
            bv, bs, be = carry
            s = st_v[pl.ds(n, _LANES)][0]
            e = en_v[pl.ds(n, _LANES)][0]

            def pos_body(k, acc):
                base = k * _LANES
                v = att_v[pl.ds(base, _LANES)]
                pos = lane + base
                m = (pos >= s) & (pos < e)
                return acc + jnp.where(m, v, 0.0)

            k0 = lax.div(s, _LANES)
            k1 = lax.div(e + (_LANES - 1), _LANES)
            acc = lax.fori_loop(k0, k1, pos_body,
                                jnp.zeros((_LANES,), jnp.float32))
            tot = jnp.sum(acc)
            upd = tot > bv
            return (jnp.where(upd, tot, bv),
                    jnp.where(upd, s, bs),
                    jnp.where(upd, e, be))

        bv, bs, be = lax.fori_loop(
            0, _N, sent_body,
            (jnp.float32(0.0), jnp.int32(0), jnp.int32(0)))
        lane2 = lax.broadcasted_iota(jnp.int32, (_LANES,), 0)
        res_v[...] = jnp.where(lane2 == 0, bs,
                               jnp.where(lane2 == 1, be, 0))
        pltpu.async_copy(res_v, out_hbm.at[pl.ds(b * _LANES, _LANES)],
                         sem).wait()


def _sc_phase1(attention, starts, ends):
    mesh = plsc.VectorSubcoreMesh(core_axis_name="c", subcore_axis_name="s")
    cp = pltpu.CompilerParams()
    if "needs_layout_passes" in pltpu.CompilerParams.__dataclass_fields__:
        cp = dataclasses.replace(cp, needs_layout_passes=False)
    kern = functools.partial(
        pl.kernel, mesh=mesh,
        compiler_params=cp,
        out_type=jax.ShapeDtypeStruct((_BATCH * _LANES,), jnp.int32),
        scratch_types=[
            pltpu.VMEM((_S,), jnp.float32),
            pltpu.VMEM((_N + _LANES,), jnp.int32),
            pltpu.VMEM((_N + _LANES,), jnp.int32),
            pltpu.VMEM((_LANES,), jnp.int32),
            pltpu.SemaphoreType.DMA,
        ],
    )(_sc_body)
    return kern(attention, starts, ends)


def _tc_body(sb_ref, ctx_ref, out_ref):
    b = pl.program_id(0)
    start = sb_ref[b, 0]
    end = sb_ref[b, 1]
    nv = end - start

    _W = _C + 8
    _G = _C // 8
    d8 = lax.rem(start, 8)
    for c in range(_NCH):
        lo = c * _C
        roff_raw = (start + lo) // 8 * 8
        clamped = roff_raw > _S - _W

        @pl.when(nv <= lo)
        def _():
            out_ref[0, lo:lo + _C, :] = jnp.zeros((_C, _D), jnp.float32)

        def _fast(masked):
            roff = pl.multiple_of(jnp.minimum(roff_raw, _S - _W), 8)
            win = ctx_ref[0, pl.ds(roff, _W), :]
            w3 = win.reshape(_W // 8, 8, _D)
            rolled = pltpu.roll(w3, lax.rem(8 - d8, 8), axis=1)
            sub = lax.broadcasted_iota(jnp.int32, (_G, 8, 1), 1)
            rot3 = jnp.where(sub < 8 - d8,
                             rolled[0:_G, :, :], rolled[1:_G + 1, :, :])
            if masked:
                grp = lax.broadcasted_iota(jnp.int32, (_G, 8, 1), 0)
                rot3 = jnp.where(grp * 8 + sub < (nv - lo), rot3, 0.0)
            out_ref[0, lo:lo + _C, :] = rot3.reshape(_C, _D)

        def _slow():
            roff = pl.multiple_of(jnp.minimum(roff_raw, _S - _W), 8)
            t = start + lo - roff
            win = ctx_ref[0, pl.ds(roff, _W), :]
            shift = lax.rem(_W - t, _W)
            rot = pltpu.roll(win, shift, axis=0)
            rows = lax.broadcasted_iota(jnp.int32, (_C, 1), 0)
            out_ref[0, lo:lo + _C, :] = jnp.where(
                rows < (nv - lo), rot[0:_C, :], 0.0)

        full = nv >= lo + _C
        tail = (nv > lo) & (nv < lo + _C)
        pl.when(full & jnp.logical_not(clamped))(lambda: _fast(False))
        pl.when(tail & jnp.logical_not(clamped))(lambda: _fast(True))
        pl.when((nv > lo) & clamped)(_slow)


def _tc_copy(sb, context):
    grid_spec = pltpu.PrefetchScalarGridSpec(
        num_scalar_prefetch=1,
        grid=(_BATCH,),
        in_specs=[
            pl.BlockSpec((1, _S, _D), lambda b, sb_ref: (b, 0, 0)),
        ],
        out_specs=pl.BlockSpec((1, _L, _D), lambda b, sb_ref: (b, 0, 0)),
    )
    return pl.pallas_call(
        _tc_body,
        grid_spec=grid_spec,
        out_shape=jax.ShapeDtypeStruct((_BATCH, _L, _D), jnp.float32),
        compiler_params=pltpu.CompilerParams(
            dimension_semantics=("arbitrary",)),
    )(sb, context)


@jax.jit
def kernel(startends, attention, context):
    starts = startends[..., 0].reshape(-1)
    ends = startends[..., 1].reshape(-1)
    att_flat = attention.reshape(-1)
    sb = _sc_phase1(att_flat, starts, ends)
    return _tc_copy(sb.reshape(_BATCH, _LANES)[:, 0:2], context)


# speedup vs baseline: 1.4043x; 1.4043x over previous
import jax
import jax.numpy as jnp
from jax.experimental import pallas as pl
from jax.experimental.pallas import tpu as pltpu

_BATCH = 16
_N = 32
_S = 2048
_L = 2048
_D = 768
_C = 256
_NCH = _L // _C
_BPS = 2


def _kern(se_ref, att_ref, ctx_ref, out_ref):
  for b2 in range(_BPS):
    att = att_ref[b2, :, :]
    starts = se_ref[b2, :, 0].reshape(_N, 1)
    ends = se_ref[b2, :, 1].reshape(_N, 1)
    pos = jax.lax.broadcasted_iota(jnp.int32, (_N, _S), 1)
    m = (pos >= starts) & (pos < ends)
    sums = jnp.sum(jnp.where(m, att, 0.0), axis=1, keepdims=True)
    maxv = jnp.max(sums)
    idx = jax.lax.broadcasted_iota(jnp.int32, (_N, 1), 0)
    best = jnp.min(jnp.where(sums == maxv, idx, _N))
    sel = maxv > 0.0
    is_best = idx == best
    start = jnp.where(sel, jnp.sum(jnp.where(is_best, starts, 0)), 0)
    end = jnp.where(sel, jnp.sum(jnp.where(is_best, ends, 0)), 0)
    nv = end - start

    _W = _C + 8
    _G = _C // 8
    d8 = jax.lax.rem(start, 8)
    for c in range(_NCH):
        lo = c * _C
        roff_raw = (start + lo) // 8 * 8
        clamped = roff_raw > _S - _W

        @pl.when(nv <= lo)
        def _():
            out_ref[b2, lo:lo + _C, :] = jnp.zeros((_C, _D), jnp.float32)

        def _fast(masked):
            roff = pl.multiple_of(jnp.minimum(roff_raw, _S - _W), 8)
            win = ctx_ref[b2, pl.ds(roff, _W), :]
            w3 = win.reshape(_W // 8, 8, _D)
            rolled = pltpu.roll(w3, jax.lax.rem(8 - d8, 8), axis=1)
            sub = jax.lax.broadcasted_iota(jnp.int32, (_G, 8, 1), 1)
            rot3 = jnp.where(sub < 8 - d8,
                             rolled[0:_G, :, :], rolled[1:_G + 1, :, :])
            if masked:
                grp = jax.lax.broadcasted_iota(jnp.int32, (_G, 8, 1), 0)
                rot3 = jnp.where(grp * 8 + sub < (nv - lo), rot3, 0.0)
            out_ref[b2, lo:lo + _C, :] = rot3.reshape(_C, _D)

        def _slow():
            roff = pl.multiple_of(jnp.minimum(roff_raw, _S - _W), 8)
            t = start + lo - roff
            win = ctx_ref[b2, pl.ds(roff, _W), :]
            shift = jax.lax.rem(_W - t, _W)
            rot = pltpu.roll(win, shift, axis=0)
            rows = jax.lax.broadcasted_iota(jnp.int32, (_C, 1), 0)
            out_ref[b2, lo:lo + _C, :] = jnp.where(
                rows < (nv - lo), rot[0:_C, :], 0.0)

        full = nv >= lo + _C
        tail = (nv > lo) & (nv < lo + _C)
        pl.when(full & jnp.logical_not(clamped))(lambda: _fast(False))
        pl.when(tail & jnp.logical_not(clamped))(lambda: _fast(True))
        pl.when((nv > lo) & clamped)(_slow)


@jax.jit
def kernel(startends, attention, context):
    att3 = attention.reshape(_BATCH, 1, _S)
    return pl.pallas_call(
        _kern,
        grid=(_BATCH // _BPS,),
        in_specs=[
            pl.BlockSpec((_BPS, _N, 2), lambda b: (b, 0, 0)),
            pl.BlockSpec((_BPS, 1, _S), lambda b: (b, 0, 0)),
            pl.BlockSpec((_BPS, _S, _D), lambda b: (b, 0, 0)),
        ],
        out_specs=pl.BlockSpec((_BPS, _L, _D), lambda b: (b, 0, 0)),
        out_shape=jax.ShapeDtypeStruct((_BATCH, _L, _D), jnp.float32),
        compiler_params=pltpu.CompilerParams(
            dimension_semantics=("parallel",)),
    )(startends, att3, context)


# speedup vs baseline: 1.4244x; 1.0143x over previous
import jax
import jax.numpy as jnp
from jax import lax
from jax.experimental import pallas as pl
from jax.experimental.pallas import tpu as pltpu

_BATCH = 16
_N = 32
_S = 2048
_L = 2048
_D = 768
_C = 256
_NCH = _L // _C
_W = _C + 8
_G = _C // 8


def _phase1(se_ref, att_ref, bb):
    att = att_ref[bb, :, :]
    starts = se_ref[bb, :, 0].reshape(_N, 1)
    ends = se_ref[bb, :, 1].reshape(_N, 1)
    pos = lax.broadcasted_iota(jnp.int32, (_N, _S), 1)
    m = (pos >= starts) & (pos < ends)
    sums = jnp.sum(jnp.where(m, att, 0.0), axis=1, keepdims=True)
    maxv = jnp.max(sums)
    idx = lax.broadcasted_iota(jnp.int32, (_N, 1), 0)
    best = jnp.min(jnp.where(sums == maxv, idx, _N))
    sel = maxv > 0.0
    is_best = idx == best
    start = jnp.where(sel, jnp.sum(jnp.where(is_best, starts, 0)), 0)
    end = jnp.where(sel, jnp.sum(jnp.where(is_best, ends, 0)), 0)
    return start, end


def _woff(start, lo):
    return pl.multiple_of(
        jnp.minimum((start + lo) // 8 * 8, _S - _W), 8)


def _issue(ctx_hbm, buf_ref, sem, bb, slot, start, end):
    nv = end - start
    for c in range(_NCH):
        lo = c * _C

        @pl.when(nv > lo)
        def _():
            pltpu.make_async_copy(
                ctx_hbm.at[bb, pl.ds(_woff(start, lo), _W), :],
                buf_ref.at[slot, c], sem,
            ).start()


def _kern(se_ref, att_ref, ctx_hbm, out_ref, buf_ref, sm_ref, sem):
    b = pl.program_id(0)
    slot = lax.rem(b, 2)
    nslot = lax.rem(b + 1, 2)

    @pl.when(b == 0)
    def _():
        s0, e0 = _phase1(se_ref, att_ref, 0)
        sm_ref[0, 0] = s0
        sm_ref[0, 1] = e0
        _issue(ctx_hbm, buf_ref, sem, 0, 0, s0, e0)

    @pl.when(b + 1 < _BATCH)
    def _():
        s1, e1 = _phase1(se_ref, att_ref, b + 1)
        sm_ref[nslot, 0] = s1
        sm_ref[nslot, 1] = e1
        _issue(ctx_hbm, buf_ref, sem, b + 1, nslot, s1, e1)

    start = sm_ref[slot, 0]
    end = sm_ref[slot, 1]
    nv = end - start
    d8 = lax.rem(start, 8)
    for c in range(_NCH):
        lo = c * _C

        @pl.when(nv <= lo)
        def _():
            out_ref[0, lo:lo + _C, :] = jnp.zeros((_C, _D), jnp.float32)

        @pl.when(nv > lo)
        def _():
            pltpu.make_async_copy(
                ctx_hbm.at[b, pl.ds(_woff(start, lo), _W), :],
                buf_ref.at[slot, c], sem,
            ).wait()

        roff_raw = (start + lo) // 8 * 8
        clamped = roff_raw > _S - _W

        def _fast(masked):
            win = buf_ref[slot, c]
            w3 = win.reshape(_W // 8, 8, _D)
            rolled = pltpu.roll(w3, lax.rem(8 - d8, 8), axis=1)
            sub = lax.broadcasted_iota(jnp.int32, (_G, 8, 1), 1)
            rot3 = jnp.where(sub < 8 - d8,
                             rolled[0:_G, :, :], rolled[1:_G + 1, :, :])
            if masked:
                grp = lax.broadcasted_iota(jnp.int32, (_G, 8, 1), 0)
                rot3 = jnp.where(grp * 8 + sub < (nv - lo), rot3, 0.0)
            out_ref[0, lo:lo + _C, :] = rot3.reshape(_C, _D)

        def _slow():
            t = start + lo - _woff(start, lo)
            win = buf_ref[slot, c]
            shift = lax.rem(_W - t, _W)
            rot = pltpu.roll(win, shift, axis=0)
            rows = lax.broadcasted_iota(jnp.int32, (_C, 1), 0)
            out_ref[0, lo:lo + _C, :] = jnp.where(
                rows < (nv - lo), rot[0:_C, :], 0.0)

        full = nv >= lo + _C
        tail = (nv > lo) & (nv < lo + _C)
        pl.when(full & jnp.logical_not(clamped))(lambda: _fast(False))
        pl.when(tail & jnp.logical_not(clamped))(lambda: _fast(True))
        pl.when((nv > lo) & clamped)(_slow)


@jax.jit
def kernel(startends, attention, context):
    att3 = attention.reshape(_BATCH, 1, _S)
    return pl.pallas_call(
        _kern,
        grid=(_BATCH,),
        in_specs=[
            pl.BlockSpec((_BATCH, _N, 2), lambda b: (0, 0, 0)),
            pl.BlockSpec((_BATCH, 1, _S), lambda b: (0, 0, 0)),
            pl.BlockSpec(memory_space=pltpu.MemorySpace.HBM),
        ],
        out_specs=pl.BlockSpec((1, _L, _D), lambda b: (b, 0, 0)),
        out_shape=jax.ShapeDtypeStruct((_BATCH, _L, _D), jnp.float32),
        scratch_shapes=[
            pltpu.VMEM((2, _NCH, _W, _D), jnp.float32),
            pltpu.SMEM((2, 2), jnp.int32),
            pltpu.SemaphoreType.DMA,
        ],
        compiler_params=pltpu.CompilerParams(
            dimension_semantics=("arbitrary",)),
    )(startends, att3, context)
